# baseline (speedup 1.0000x reference)
import functools

import numpy as np
import jax
import jax.numpy as jnp
from jax import lax
from jax.experimental import pallas as pl
from jax.experimental.pallas import tpu as pltpu
from jax.experimental.pallas import tpu_sc as plsc

_N_BINS = 15
_BLOCK_R = 16384

_bounds = np.linspace(0.0, 1.0, _N_BINS + 1, dtype=np.float32)
_UPPERS = [float(v) for v in _bounds[1:15]]


def _rows_kernel(x_ref, lab_ref, conf_ref, acc_ref):
    x = x_ref[...]
    c = x.shape[1]
    m = jnp.max(x, axis=1, keepdims=True)
    ez = jnp.exp(x - m)
    eqb = (x == m).astype(jnp.bfloat16)

    ez_hi = ez.astype(jnp.bfloat16)
    ez_lo = (ez - ez_hi.astype(jnp.float32)).astype(jnp.bfloat16)
    ones_row = jnp.ones((1, c), jnp.bfloat16)
    iota_row = lax.broadcasted_iota(jnp.int32, (1, c), 1).astype(jnp.bfloat16)
    dn = (((1,), (1,)), ((), ()))
    s = (lax.dot_general(ones_row, ez_hi, dn, preferred_element_type=jnp.float32)
         + lax.dot_general(ones_row, ez_lo, dn, preferred_element_type=jnp.float32))
    pred = lax.dot_general(iota_row, eqb, dn, preferred_element_type=jnp.float32)
    conf_ref[...] = (1.0 / s).reshape(conf_ref.shape)
    lab = lab_ref[0].astype(jnp.float32)
    acc_ref[...] = (pred == lab).astype(jnp.float32).reshape(acc_ref.shape)


def _bin_kernel(conf_hbm, acc_hbm, out_hbm, cbuf, abuf, bidx, obuf,
                cnt_sh, cf_sh, ac_sh):
    info = plsc.get_sparse_core_info()
    nw = info.num_cores * info.num_subcores
    sid = lax.axis_index("s")
    cid = lax.axis_index("c")
    wid = sid * info.num_cores + cid
    rpw = conf_hbm.shape[0] // nw
    base = wid * rpw
    pltpu.sync_copy(conf_hbm.at[pl.ds(base, rpw)], cbuf)
    pltpu.sync_copy(acc_hbm.at[pl.ds(base, rpw)], abuf)

    obuf[...] = jnp.zeros((16,), jnp.float32)

    @pl.when(sid == 0)
    def _init_shared():
        pltpu.sync_copy(obuf, cnt_sh)
        pltpu.sync_copy(obuf, cf_sh)
        pltpu.sync_copy(obuf, ac_sh)

    plsc.subcore_barrier()
    obuf[...] = jnp.ones((16,), jnp.float32)

    @pl.loop(0, rpw // 16)
    def _per_group(g):
        conf = cbuf[pl.ds(g * 16, 16)]
        b = jnp.zeros((16,), jnp.int32)
        for u in _UPPERS:
            b = b + jnp.where(conf > u, 1, 0).astype(jnp.int32)
        bidx[0, pl.ds(g * 16, 16)] = b

    @pl.loop(0, rpw // 16)
    def _per_scatter(g):
        idx = bidx.at[0, pl.ds(g * 16, 16)]
        pltpu.sync_copy(obuf, cnt_sh.at[idx], add=True)
        pltpu.sync_copy(cbuf.at[pl.ds(g * 16, 16)], cf_sh.at[idx], add=True)
        pltpu.sync_copy(abuf.at[pl.ds(g * 16, 16)], ac_sh.at[idx], add=True)

    plsc.subcore_barrier()

    @pl.when(sid == 0)
    def _flush():
        pltpu.sync_copy(cnt_sh, obuf)
        pltpu.sync_copy(obuf, out_hbm.at[pl.ds(cid * 48, 16)])
        pltpu.sync_copy(cf_sh, obuf)
        pltpu.sync_copy(obuf, out_hbm.at[pl.ds(cid * 48 + 16, 16)])
        pltpu.sync_copy(ac_sh, obuf)
        pltpu.sync_copy(obuf, out_hbm.at[pl.ds(cid * 48 + 32, 16)])


def _combine_kernel(st_ref, out_ref, *, inv_n):
    st = st_ref[...]
    tot = jnp.sum(st, axis=0, keepdims=True)
    cb = tot[:, 0:15]
    sconf = tot[:, 16:31]
    sacc = tot[:, 32:47]
    safe = jnp.maximum(cb, 1.0)
    contrib = jnp.abs(sconf - sacc) / safe * (cb * inv_n)
    contrib = jnp.where(cb > 0.0, contrib, 0.0)
    out_ref[...] = jnp.sum(contrib, axis=1, keepdims=True)


def kernel(logits_input, labels_input):
    n, c = logits_input.shape
    grid = n // _BLOCK_R
    labels = labels_input.astype(jnp.int32).reshape(grid, 1, _BLOCK_R)
    conf, acc = pl.pallas_call(
        _rows_kernel,
        grid=(grid,),
        in_specs=[
            pl.BlockSpec((_BLOCK_R, c), lambda i: (i, 0)),
            pl.BlockSpec((1, 1, _BLOCK_R), lambda i: (i, 0, 0)),
        ],
        out_specs=[pl.BlockSpec((1, 1, _BLOCK_R), lambda i: (i, 0, 0)),
                   pl.BlockSpec((1, 1, _BLOCK_R), lambda i: (i, 0, 0))],
        out_shape=[jax.ShapeDtypeStruct((grid, 1, _BLOCK_R), jnp.float32),
                   jax.ShapeDtypeStruct((grid, 1, _BLOCK_R), jnp.float32)],
    )(logits_input, labels)

    info = plsc.get_sparse_core_info()
    nw = info.num_cores * info.num_subcores
    rpw = n // nw
    mesh = plsc.VectorSubcoreMesh(core_axis_name="c", subcore_axis_name="s", num_cores=info.num_cores)
    stats = pl.kernel(
        _bin_kernel,
        mesh=mesh,
        out_type=jax.ShapeDtypeStruct((info.num_cores * 48,), jnp.float32),
        scratch_types=[
            pltpu.VMEM((rpw,), jnp.float32),
            pltpu.VMEM((rpw,), jnp.float32),
            pltpu.VMEM((1, rpw), jnp.int32),
            pltpu.VMEM((16,), jnp.float32),
            pltpu.VMEM_SHARED((16,), jnp.float32),
            pltpu.VMEM_SHARED((16,), jnp.float32),
            pltpu.VMEM_SHARED((16,), jnp.float32),
        ],
    )(conf.reshape(n), acc.reshape(n))

    out = pl.pallas_call(
        functools.partial(_combine_kernel, inv_n=1.0 / n),
        grid=(1,),
        in_specs=[pl.BlockSpec((2, 48), lambda i: (0, 0))],
        out_specs=pl.BlockSpec((1, 1), lambda i: (0, 0)),
        out_shape=jax.ShapeDtypeStruct((1, 1), jnp.float32),
    )(stats.reshape(info.num_cores, 48))
    return out.reshape(1)


# speedup vs baseline: 1.7897x; 1.7897x over previous
import functools

import numpy as np
import jax
import jax.numpy as jnp
from jax.experimental import pallas as pl
from jax.experimental.pallas import tpu as pltpu

_N_BINS = 15
_BLOCK_R = 16384

_bounds = np.linspace(0.0, 1.0, _N_BINS + 1, dtype=np.float32)
_UP_COL = np.full((16, 1), 2.0, dtype=np.float32)
_UP_COL[:14, 0] = _bounds[1:15]
_UP_COL[14, 0] = -1.0


def _ece_block_kernel(x_ref, lab_ref, up_ref, out_ref,
                      cnt_ref, cf_ref, ac_ref, *, inv_n):
    i = pl.program_id(0)

    @pl.when(i == 0)
    def _init():
        cnt_ref[...] = jnp.zeros_like(cnt_ref)
        cf_ref[...] = jnp.zeros_like(cf_ref)
        ac_ref[...] = jnp.zeros_like(ac_ref)

    x = x_ref[...]
    c = x.shape[1]
    m = jnp.max(x, axis=1, keepdims=True)
    ez = jnp.exp(x - m)
    eqb = (x == m).astype(jnp.bfloat16)

    ez_hi = ez.astype(jnp.bfloat16)
    ez_lo = (ez - ez_hi.astype(jnp.float32)).astype(jnp.bfloat16)
    ones_row = jnp.ones((1, c), jnp.bfloat16)
    iota_row = jax.lax.broadcasted_iota(jnp.int32, (1, c), 1).astype(jnp.bfloat16)
    dn = (((1,), (1,)), ((), ()))
    s_hi = jax.lax.dot_general(ones_row, ez_hi, dn,
                               preferred_element_type=jnp.float32)
    s_lo = jax.lax.dot_general(ones_row, ez_lo, dn,
                               preferred_element_type=jnp.float32)
    s = s_hi + s_lo
    pred = jax.lax.dot_general(iota_row, eqb, dn,
                               preferred_element_type=jnp.float32)
    conf = 1.0 / s
    lab = lab_ref[0].astype(jnp.float32)
    acc = (pred == lab).astype(jnp.float32)

    mask = (conf > up_ref[...]).astype(jnp.float32)
    cnt_ref[...] += mask
    cf_ref[...] += mask * conf
    ac_ref[...] += mask * acc

    @pl.when(i == pl.num_programs(0) - 1)
    def _fini():
        cum = jnp.concatenate(
            [jnp.sum(cnt_ref[...], axis=1, keepdims=True),
             jnp.sum(cf_ref[...], axis=1, keepdims=True),
             jnp.sum(ac_ref[...], axis=1, keepdims=True)], axis=1)
        total = cum[14:15, :]
        prev = jnp.concatenate([total, cum[0:14, :]], axis=0)
        cur = jnp.concatenate(
            [cum[0:14, :], jnp.zeros((1, 3), jnp.float32)], axis=0)
        stats = prev - cur
        cb = stats[:, 0:1]
        safe = jnp.maximum(cb, 1.0)
        contrib = jnp.abs(stats[:, 1:2] - stats[:, 2:3]) / safe * (cb * inv_n)
        contrib = jnp.where(cb > 0.0, contrib, 0.0)
        out_ref[...] = jnp.sum(contrib, axis=0, keepdims=True)


def kernel(logits_input, labels_input):
    n, c = logits_input.shape
    grid = n // _BLOCK_R
    labels = labels_input.astype(jnp.int32).reshape(grid, 1, _BLOCK_R)
    out = pl.pallas_call(
        functools.partial(_ece_block_kernel, inv_n=1.0 / n),
        grid=(grid,),
        in_specs=[
            pl.BlockSpec((_BLOCK_R, c), lambda i: (i, 0)),
            pl.BlockSpec((1, 1, _BLOCK_R), lambda i: (i, 0, 0)),
            pl.BlockSpec((16, 1), lambda i: (0, 0)),
        ],
        out_specs=pl.BlockSpec((1, 1), lambda i: (0, 0)),
        out_shape=jax.ShapeDtypeStruct((1, 1), jnp.float32),
        scratch_shapes=[pltpu.VMEM((16, _BLOCK_R), jnp.float32),
                        pltpu.VMEM((16, _BLOCK_R), jnp.float32),
                        pltpu.VMEM((16, _BLOCK_R), jnp.float32)],
    )(logits_input, labels, jnp.asarray(_UP_COL))
    return out.reshape(1)
